# Initial kernel scaffold; baseline (speedup 1.0000x reference)
import jax
import jax.numpy as jnp
from jax.experimental import pallas as pl


def kernel(x, W, b):
    raise NotImplementedError("write your pallas kernel here")


# speedup vs baseline: 1.1719x; 1.1719x over previous
import functools

import jax
import jax.numpy as jnp
from jax.experimental import pallas as pl
from jax.experimental.pallas import tpu as pltpu

_NUM_EXPERTS = 16
_TOP_K = 2
_LANES = 128
_NEG = -1e30


def _router_body(num_tiles, x_ref, w_ref, b_ref, idx_ref, wgt_ref, loss_ref,
                 cnt_ref):
    step = pl.program_id(0)
    tile = x_ref.shape[0]

    logits = jax.lax.dot_general(
        x_ref[...], w_ref[...], (((1,), (0,)), ((), ())),
        preferred_element_type=jnp.float32) + b_ref[...]
    lane = jax.lax.broadcasted_iota(jnp.int32, (tile, _LANES), 1)
    logits = jnp.where(lane < _NUM_EXPERTS, logits, _NEG)

    m1 = jnp.max(logits, axis=1, keepdims=True)
    i1 = jnp.min(jnp.where(logits == m1, lane, _LANES), axis=1, keepdims=True)
    rest = jnp.where(lane == i1, _NEG, logits)
    m2 = jnp.max(rest, axis=1, keepdims=True)
    i2 = jnp.min(jnp.where(rest == m2, lane, _LANES), axis=1, keepdims=True)

    e2 = jnp.exp(m2 - m1)
    w1 = 1.0 / (1.0 + e2)

    idx_ref[...] = jnp.concatenate([i1, i2], axis=1)
    wgt_ref[...] = jnp.concatenate([w1, 1.0 - w1], axis=1)

    c = (jnp.sum(jnp.where(lane == i1, 1.0, 0.0), axis=0, keepdims=True) +
         jnp.sum(jnp.where(lane == i2, 1.0, 0.0), axis=0, keepdims=True))

    @pl.when(step == 0)
    def _():
        cnt_ref[...] = c

    @pl.when(step > 0)
    def _():
        cnt_ref[...] = cnt_ref[...] + c

    @pl.when(step == num_tiles - 1)
    def _():
        cnts = cnt_ref[...]
        mean = jnp.sum(cnts) / _NUM_EXPERTS
        emask = lane[0:1, :] < _NUM_EXPERTS
        var = jnp.sum(jnp.where(emask, (cnts - mean) ** 2, 0.0)) / (
            _NUM_EXPERTS - 1)
        loss_ref[...] = jnp.reshape(jnp.sqrt(var) / (mean + 1e-10) * 0.01,
                                    (1, 1))


@functools.partial(jax.jit, static_argnames=())
def kernel(x, W, b):
    B, S, D = x.shape
    T = B * S
    xf = x.reshape(T, D)

    Wp = jnp.zeros((D, _LANES), jnp.float32).at[:, :_NUM_EXPERTS].set(W)
    bp = jnp.zeros((1, _LANES), jnp.float32).at[0, :_NUM_EXPERTS].set(b)

    tile = 512
    num_tiles = T // tile

    idx, wgt, loss = pl.pallas_call(
        functools.partial(_router_body, num_tiles),
        grid=(num_tiles,),
        in_specs=[
            pl.BlockSpec((tile, D), lambda i: (i, 0)),
            pl.BlockSpec((D, _LANES), lambda i: (0, 0)),
            pl.BlockSpec((1, _LANES), lambda i: (0, 0)),
        ],
        out_specs=[
            pl.BlockSpec((tile, _TOP_K), lambda i: (i, 0)),
            pl.BlockSpec((tile, _TOP_K), lambda i: (i, 0)),
            pl.BlockSpec((1, 1), lambda i: (0, 0)),
        ],
        out_shape=[
            jax.ShapeDtypeStruct((T, _TOP_K), jnp.int32),
            jax.ShapeDtypeStruct((T, _TOP_K), jnp.float32),
            jax.ShapeDtypeStruct((1, 1), jnp.float32),
        ],
        scratch_shapes=[pltpu.VMEM((1, _LANES), jnp.float32)],
    )(xf, Wp, bp)

    return (idx.reshape(B, S, _TOP_K), wgt.reshape(B, S, _TOP_K), loss[0, 0])


# speedup vs baseline: 1.3557x; 1.1569x over previous
import functools

import jax
import jax.numpy as jnp
from jax.experimental import pallas as pl
from jax.experimental.pallas import tpu as pltpu

_NUM_EXPERTS = 16
_TOP_K = 2
_LANES = 128
_NEG = -1e30


def _router_body(num_tiles, x_ref, w_ref, b_ref, idx_ref, wgt_ref, loss_ref,
                 cnt_ref):
    step = pl.program_id(0)
    tile = x_ref.shape[0]

    logits = jax.lax.dot_general(
        x_ref[...], w_ref[...], (((1,), (0,)), ((), ())),
        preferred_element_type=jnp.float32) + b_ref[...]
    lane = jax.lax.broadcasted_iota(jnp.int32, (tile, _LANES), 1)
    logits = jnp.where(lane < _NUM_EXPERTS, logits, _NEG)

    m1 = jnp.max(logits, axis=1, keepdims=True)
    i1 = jnp.min(jnp.where(logits == m1, lane, _LANES), axis=1, keepdims=True)
    rest = jnp.where(lane == i1, _NEG, logits)
    m2 = jnp.max(rest, axis=1, keepdims=True)
    i2 = jnp.min(jnp.where(rest == m2, lane, _LANES), axis=1, keepdims=True)

    e2 = jnp.exp(m2 - m1)
    w1 = 1.0 / (1.0 + e2)

    idx_ref[...] = jnp.concatenate([i1, i2], axis=1)
    wgt_ref[...] = jnp.concatenate([w1, 1.0 - w1], axis=1)

    c = (jnp.sum(jnp.where(lane == i1, 1.0, 0.0), axis=0, keepdims=True) +
         jnp.sum(jnp.where(lane == i2, 1.0, 0.0), axis=0, keepdims=True))

    @pl.when(step == 0)
    def _():
        cnt_ref[...] = c

    @pl.when(step > 0)
    def _():
        cnt_ref[...] = cnt_ref[...] + c

    @pl.when(step == num_tiles - 1)
    def _():
        cnts = cnt_ref[...]
        mean = jnp.sum(cnts) / _NUM_EXPERTS
        emask = lane[0:1, :] < _NUM_EXPERTS
        var = jnp.sum(jnp.where(emask, (cnts - mean) ** 2, 0.0)) / (
            _NUM_EXPERTS - 1)
        loss_ref[...] = jnp.reshape(jnp.sqrt(var) / (mean + 1e-10) * 0.01,
                                    (1, 1))


@functools.partial(jax.jit, static_argnames=())
def kernel(x, W, b):
    B, S, D = x.shape
    T = B * S
    xf = x.reshape(T, D)

    Wp = jnp.zeros((D, _LANES), jnp.float32).at[:, :_NUM_EXPERTS].set(W)
    bp = jnp.zeros((1, _LANES), jnp.float32).at[0, :_NUM_EXPERTS].set(b)

    tile = 1024
    num_tiles = T // tile

    idx, wgt, loss = pl.pallas_call(
        functools.partial(_router_body, num_tiles),
        grid=(num_tiles,),
        in_specs=[
            pl.BlockSpec((tile, D), lambda i: (i, 0)),
            pl.BlockSpec((D, _LANES), lambda i: (0, 0)),
            pl.BlockSpec((1, _LANES), lambda i: (0, 0)),
        ],
        out_specs=[
            pl.BlockSpec((tile, _TOP_K), lambda i: (i, 0)),
            pl.BlockSpec((tile, _TOP_K), lambda i: (i, 0)),
            pl.BlockSpec((1, 1), lambda i: (0, 0)),
        ],
        out_shape=[
            jax.ShapeDtypeStruct((T, _TOP_K), jnp.int32),
            jax.ShapeDtypeStruct((T, _TOP_K), jnp.float32),
            jax.ShapeDtypeStruct((1, 1), jnp.float32),
        ],
        scratch_shapes=[pltpu.VMEM((1, _LANES), jnp.float32)],
    )(xf, Wp, bp)

    return (idx.reshape(B, S, _TOP_K), wgt.reshape(B, S, _TOP_K), loss[0, 0])


# speedup vs baseline: 1.4186x; 1.0464x over previous
import functools

import jax
import jax.numpy as jnp
from jax.experimental import pallas as pl
from jax.experimental.pallas import tpu as pltpu

_NUM_EXPERTS = 16
_TOP_K = 2
_LANES = 128
_NEG = -1e30


def _router_body(num_tiles, x_ref, w_ref, b_ref, idx_ref, wgt_ref, loss_ref,
                 cnt_ref):
    step = pl.program_id(0)
    tile = x_ref.shape[0]

    logits = jax.lax.dot_general(
        x_ref[...], w_ref[...], (((1,), (0,)), ((), ())),
        preferred_element_type=jnp.float32) + b_ref[...]
    lane = jax.lax.broadcasted_iota(jnp.int32, (tile, _LANES), 1)
    logits = jnp.where(lane < _NUM_EXPERTS, logits, _NEG)

    m1 = jnp.max(logits, axis=1, keepdims=True)
    i1 = jnp.min(jnp.where(logits == m1, lane, _LANES), axis=1, keepdims=True)
    rest = jnp.where(lane == i1, _NEG, logits)
    m2 = jnp.max(rest, axis=1, keepdims=True)
    i2 = jnp.min(jnp.where(rest == m2, lane, _LANES), axis=1, keepdims=True)

    e2 = jnp.exp(m2 - m1)
    w1 = 1.0 / (1.0 + e2)

    idx_ref[...] = jnp.concatenate([i1, i2], axis=1)
    wgt_ref[...] = jnp.concatenate([w1, 1.0 - w1], axis=1)

    c = (jnp.sum(jnp.where(lane == i1, 1.0, 0.0), axis=0, keepdims=True) +
         jnp.sum(jnp.where(lane == i2, 1.0, 0.0), axis=0, keepdims=True))

    @pl.when(step == 0)
    def _():
        cnt_ref[...] = c

    @pl.when(step > 0)
    def _():
        cnt_ref[...] = cnt_ref[...] + c

    @pl.when(step == num_tiles - 1)
    def _():
        cnts = cnt_ref[...]
        mean = jnp.sum(cnts) / _NUM_EXPERTS
        emask = lane[0:1, :] < _NUM_EXPERTS
        var = jnp.sum(jnp.where(emask, (cnts - mean) ** 2, 0.0)) / (
            _NUM_EXPERTS - 1)
        loss_ref[...] = jnp.reshape(jnp.sqrt(var) / (mean + 1e-10) * 0.01,
                                    (1, 1))


@functools.partial(jax.jit, static_argnames=())
def kernel(x, W, b):
    B, S, D = x.shape
    T = B * S
    xf = x.reshape(T, D)

    Wp = jnp.zeros((D, _LANES), jnp.float32).at[:, :_NUM_EXPERTS].set(W)
    bp = jnp.zeros((1, _LANES), jnp.float32).at[0, :_NUM_EXPERTS].set(b)

    tile = 2048
    num_tiles = T // tile

    idx, wgt, loss = pl.pallas_call(
        functools.partial(_router_body, num_tiles),
        grid=(num_tiles,),
        in_specs=[
            pl.BlockSpec((tile, D), lambda i: (i, 0)),
            pl.BlockSpec((D, _LANES), lambda i: (0, 0)),
            pl.BlockSpec((1, _LANES), lambda i: (0, 0)),
        ],
        out_specs=[
            pl.BlockSpec((tile, _TOP_K), lambda i: (i, 0)),
            pl.BlockSpec((tile, _TOP_K), lambda i: (i, 0)),
            pl.BlockSpec((1, 1), lambda i: (0, 0)),
        ],
        out_shape=[
            jax.ShapeDtypeStruct((T, _TOP_K), jnp.int32),
            jax.ShapeDtypeStruct((T, _TOP_K), jnp.float32),
            jax.ShapeDtypeStruct((1, 1), jnp.float32),
        ],
        scratch_shapes=[pltpu.VMEM((1, _LANES), jnp.float32)],
    )(xf, Wp, bp)

    return (idx.reshape(B, S, _TOP_K), wgt.reshape(B, S, _TOP_K), loss[0, 0])
